# initial kernel scaffold (unmeasured)
import jax
import jax.numpy as jnp
from jax import lax
from jax.experimental import pallas as pl
from jax.experimental.pallas import tpu as pltpu

N_DEV = 32
N_STAGES = 5
E_PER = 2
N_TOK = 256
N_EXP = 64
H = 256


def kernel(x, router_W, route_idx, expert_W, shared_W):
    def body(x_ref, rw_ref, idx_ref, ew_ref, sw_ref, out_ref,
             acc_ref, comm_ref, send_sems, recv_sems):
        my = lax.axis_index("i")

        xb = x_ref[:, :].astype(jnp.bfloat16)
        scores = jnp.dot(xb, rw_ref[:, :].astype(jnp.bfloat16),
                         preferred_element_type=jnp.float32)
        s_max = jnp.max(scores, axis=-1, keepdims=True)
        p = jnp.exp(scores - s_max)
        probs = p / jnp.sum(p, axis=-1, keepdims=True)

        col = lax.broadcasted_iota(jnp.int32, (N_TOK, N_EXP), 1)
        routed = col == idx_ref[:, :]

        partial = jnp.zeros((N_TOK, H), jnp.float32)
        for k in range(E_PER):
            e = my * E_PER + k
            w = jnp.sum(jnp.where(routed & (col == e), probs, 0.0),
                        axis=1, keepdims=True)
            ye = jnp.dot(xb, ew_ref[k].astype(jnp.bfloat16),
                         preferred_element_type=jnp.float32)
            partial = partial + w * ye
        acc_ref[:, :] = partial.astype(jnp.bfloat16)

        for s in range(N_STAGES):
            partner = jnp.bitwise_xor(my, 1 << s)
            rdma = pltpu.make_async_remote_copy(
                src_ref=acc_ref,
                dst_ref=comm_ref.at[s],
                send_sem=send_sems.at[s],
                recv_sem=recv_sems.at[s],
                device_id=(partner,),
                device_id_type=pl.DeviceIdType.MESH,
            )
            rdma.start()
            rdma.wait()
            acc_ref[:, :] = acc_ref[:, :] + comm_ref[s]

        shared = jnp.dot(xb, sw_ref[:, :].astype(jnp.bfloat16),
                         preferred_element_type=jnp.float32)
        out_ref[:, :] = acc_ref[:, :].astype(jnp.float32) + shared

    return pl.pallas_call(
        body,
        out_shape=jax.ShapeDtypeStruct((N_TOK, H), jnp.float32),
        in_specs=[pl.BlockSpec(memory_space=pltpu.VMEM)] * 5,
        out_specs=pl.BlockSpec(memory_space=pltpu.VMEM),
        scratch_shapes=[
            pltpu.VMEM((N_TOK, H), jnp.bfloat16),
            pltpu.VMEM((N_STAGES, N_TOK, H), jnp.bfloat16),
            pltpu.SemaphoreType.DMA((N_STAGES,)),
            pltpu.SemaphoreType.DMA((N_STAGES,)),
        ],
        compiler_params=pltpu.CompilerParams(collective_id=0),
    )(x, router_W, route_idx, expert_W, shared_W)


# baseline (device time: 37233 ns/iter reference)
import jax
import jax.numpy as jnp
from jax import lax
from jax.experimental import pallas as pl
from jax.experimental.pallas import tpu as pltpu

N_DEV = 32
N_STAGES = 5
E_PER = 2
N_TOK = 256
N_EXP = 64
H = 256


def kernel(x, router_W, route_idx, expert_W, shared_W):
    def body(x_ref, rw_ref, idx_ref, ew_ref, sw_ref, out_ref,
             acc_ref, comm_ref, send_sems, recv_sems):
        my = lax.axis_index("i")

        xb = x_ref[:, :].astype(jnp.bfloat16)
        scores = jnp.dot(xb, rw_ref[:, :].astype(jnp.bfloat16),
                         preferred_element_type=jnp.float32)
        s_max = jnp.max(scores, axis=-1, keepdims=True)
        p = jnp.exp(scores - s_max)
        probs = p / jnp.sum(p, axis=-1, keepdims=True)

        col = lax.broadcasted_iota(jnp.int32, (N_TOK, N_EXP), 1)
        routed = col == idx_ref[:, :]

        partial = jnp.zeros((N_TOK, H), jnp.float32)
        for k in range(E_PER):
            e = my * E_PER + k
            w = jnp.sum(jnp.where(routed & (col == e), probs, 0.0),
                        axis=1, keepdims=True)
            ye = jnp.dot(xb, ew_ref[k].astype(jnp.bfloat16),
                         preferred_element_type=jnp.float32)
            partial = partial + w * ye
        acc_ref[:, :] = partial.astype(jnp.bfloat16)

        for s in range(N_STAGES):
            partner = jnp.bitwise_xor(my, 1 << s)
            rdma = pltpu.make_async_remote_copy(
                src_ref=acc_ref,
                dst_ref=comm_ref.at[s],
                send_sem=send_sems.at[s],
                recv_sem=recv_sems.at[s],
                device_id=(partner,),
                device_id_type=pl.DeviceIdType.MESH,
            )
            rdma.start()
            rdma.wait()
            acc_ref[:, :] = acc_ref[:, :] + comm_ref[s]

        shared = jnp.dot(xb, sw_ref[:, :].astype(jnp.bfloat16),
                         preferred_element_type=jnp.float32)
        out_ref[:, :] = acc_ref[:, :].astype(jnp.float32) + shared

    return pl.pallas_call(
        body,
        out_shape=jax.ShapeDtypeStruct((N_TOK, H), jnp.float32),
        in_specs=[pl.BlockSpec(memory_space=pltpu.VMEM)] * 5,
        out_specs=pl.BlockSpec(memory_space=pltpu.VMEM),
        scratch_shapes=[
            pltpu.VMEM((N_TOK, H), jnp.bfloat16),
            pltpu.VMEM((N_STAGES, N_TOK, H), jnp.bfloat16),
            pltpu.SemaphoreType.DMA((N_STAGES,)),
            pltpu.SemaphoreType.DMA((N_STAGES,)),
        ],
    )(x, router_W, route_idx, expert_W, shared_W)


# device time: 29704 ns/iter; 1.2535x vs baseline; 1.2535x over previous
import jax
import jax.numpy as jnp
from jax import lax
from jax.experimental import pallas as pl
from jax.experimental.pallas import tpu as pltpu

N_DEV = 32
N_STAGES = 5
E_PER = 2
N_TOK = 256
N_EXP = 64
H = 256


def kernel(x, router_W, route_idx, expert_W, shared_W):
    def body(x_ref, rw_ref, idx_ref, ew_ref, sw_ref, out_ref,
             acc_ref, comm_ref, send_sems, recv_sems):
        my = lax.axis_index("i")

        barrier_sem = pltpu.get_barrier_semaphore()
        for s in range(N_STAGES):
            partner = jnp.bitwise_xor(my, 1 << s)
            pl.semaphore_signal(barrier_sem, inc=1, device_id=(partner,),
                                device_id_type=pl.DeviceIdType.MESH)
        pl.semaphore_wait(barrier_sem, N_STAGES)

        xb = x_ref[:, :].astype(jnp.bfloat16)
        scores = jnp.dot(xb, rw_ref[:, :].astype(jnp.bfloat16),
                         preferred_element_type=jnp.float32)
        s_max = jnp.max(scores, axis=-1, keepdims=True)
        p = jnp.exp(scores - s_max)
        probs = p / jnp.sum(p, axis=-1, keepdims=True)

        col = lax.broadcasted_iota(jnp.int32, (N_TOK, N_EXP), 1)
        routed = col == idx_ref[:, :]

        partial = jnp.zeros((N_TOK, H), jnp.float32)
        for k in range(E_PER):
            e = my * E_PER + k
            w = jnp.sum(jnp.where(routed & (col == e), probs, 0.0),
                        axis=1, keepdims=True)
            ye = jnp.dot(xb, ew_ref[k].astype(jnp.bfloat16),
                         preferred_element_type=jnp.float32)
            partial = partial + w * ye
        acc_ref[0] = partial.astype(jnp.bfloat16)

        rdmas = []
        for s in range(N_STAGES):
            partner = jnp.bitwise_xor(my, 1 << s)
            rdma = pltpu.make_async_remote_copy(
                src_ref=acc_ref.at[s],
                dst_ref=comm_ref.at[s],
                send_sem=send_sems.at[s],
                recv_sem=recv_sems.at[s],
                device_id=(partner,),
                device_id_type=pl.DeviceIdType.MESH,
            )
            rdma.start()
            rdmas.append(rdma)
            if s == 0:
                shared = jnp.dot(xb, sw_ref[:, :].astype(jnp.bfloat16),
                                 preferred_element_type=jnp.float32)
            rdma.wait_recv()
            acc_ref[s + 1] = acc_ref[s] + comm_ref[s]

        out_ref[:, :] = acc_ref[N_STAGES].astype(jnp.float32) + shared
        for rdma in rdmas:
            rdma.wait_send()

    return pl.pallas_call(
        body,
        out_shape=jax.ShapeDtypeStruct((N_TOK, H), jnp.float32),
        in_specs=[pl.BlockSpec(memory_space=pltpu.VMEM)] * 5,
        out_specs=pl.BlockSpec(memory_space=pltpu.VMEM),
        scratch_shapes=[
            pltpu.VMEM((N_STAGES + 1, N_TOK, H), jnp.bfloat16),
            pltpu.VMEM((N_STAGES, N_TOK, H), jnp.bfloat16),
            pltpu.SemaphoreType.DMA((N_STAGES,)),
            pltpu.SemaphoreType.DMA((N_STAGES,)),
        ],
        compiler_params=pltpu.CompilerParams(collective_id=0),
    )(x, router_W, route_idx, expert_W, shared_W)


# device time: 28517 ns/iter; 1.3056x vs baseline; 1.0416x over previous
import jax
import jax.numpy as jnp
from jax import lax
from jax.experimental import pallas as pl
from jax.experimental.pallas import tpu as pltpu

N_DEV = 32
N_STAGES = 5
MASKS = (1, 3, 8, 4, 16)
E_PER = 2
N_TOK = 256
N_EXP = 64
H = 256


def kernel(x, router_W, route_idx, expert_W, shared_W):
    def body(x_ref, rw_ref, idx_ref, ew_ref, sw_ref, out_ref,
             acc_ref, comm_ref, send_sems, recv_sems):
        my = lax.axis_index("i")

        barrier_sem = pltpu.get_barrier_semaphore()
        for m in MASKS:
            partner = jnp.bitwise_xor(my, m)
            pl.semaphore_signal(barrier_sem, inc=1, device_id=(partner,),
                                device_id_type=pl.DeviceIdType.MESH)

        xb = x_ref[:, :].astype(jnp.bfloat16)
        scores = jnp.dot(xb, rw_ref[:, :].astype(jnp.bfloat16),
                         preferred_element_type=jnp.float32)
        s_max = jnp.max(scores, axis=-1, keepdims=True)
        p = jnp.exp(scores - s_max)
        probs = p / jnp.sum(p, axis=-1, keepdims=True)

        col = lax.broadcasted_iota(jnp.int32, (N_TOK, N_EXP), 1)
        routed = col == idx_ref[:, :]

        partial = jnp.zeros((N_TOK, H), jnp.float32)
        for k in range(E_PER):
            e = my * E_PER + k
            w = jnp.sum(jnp.where(routed & (col == e), probs, 0.0),
                        axis=1, keepdims=True)
            ye = jnp.dot(xb, ew_ref[k].astype(jnp.bfloat16),
                         preferred_element_type=jnp.float32)
            partial = partial + w * ye
        acc_ref[0] = partial.astype(jnp.bfloat16)

        pl.semaphore_wait(barrier_sem, N_STAGES)

        rdmas = []
        for s, m in enumerate(MASKS):
            partner = jnp.bitwise_xor(my, m)
            rdma = pltpu.make_async_remote_copy(
                src_ref=acc_ref.at[s],
                dst_ref=comm_ref.at[s],
                send_sem=send_sems.at[s],
                recv_sem=recv_sems.at[s],
                device_id=(partner,),
                device_id_type=pl.DeviceIdType.MESH,
            )
            rdma.start()
            rdmas.append(rdma)
            if s == 0:
                shared = jnp.dot(xb, sw_ref[:, :].astype(jnp.bfloat16),
                                 preferred_element_type=jnp.float32)
            rdma.wait_recv()
            acc_ref[s + 1] = acc_ref[s] + comm_ref[s]

        out_ref[:, :] = acc_ref[N_STAGES].astype(jnp.float32) + shared
        for rdma in rdmas:
            rdma.wait_send()

    return pl.pallas_call(
        body,
        out_shape=jax.ShapeDtypeStruct((N_TOK, H), jnp.float32),
        in_specs=[pl.BlockSpec(memory_space=pltpu.VMEM)] * 5,
        out_specs=pl.BlockSpec(memory_space=pltpu.VMEM),
        scratch_shapes=[
            pltpu.VMEM((N_STAGES + 1, N_TOK, H), jnp.bfloat16),
            pltpu.VMEM((N_STAGES, N_TOK, H), jnp.bfloat16),
            pltpu.SemaphoreType.DMA((N_STAGES,)),
            pltpu.SemaphoreType.DMA((N_STAGES,)),
        ],
        compiler_params=pltpu.CompilerParams(collective_id=0),
    )(x, router_W, route_idx, expert_W, shared_W)


# device time: 23780 ns/iter; 1.5657x vs baseline; 1.1992x over previous
import jax
import jax.numpy as jnp
from jax import lax
from jax.experimental import pallas as pl
from jax.experimental.pallas import tpu as pltpu

N_DEV = 32
N_STAGES = 5
MASKS_A = (8, 1, 4, 16, 3)
MASKS_B = (3, 16, 1, 4, 8)
E_PER = 2
N_TOK = 256
N_EXP = 64
H = 256
H_HALF = H // 2


def kernel(x, router_W, route_idx, expert_W, shared_W):
    def body(x_ref, rw_ref, idx_ref, ew_ref, sw_ref, out_ref,
             acc_a, acc_b, comm_a, comm_b,
             send_a, recv_a, send_b, recv_b):
        my = lax.axis_index("i")

        barrier_sem = pltpu.get_barrier_semaphore()
        for m in MASKS_A:
            partner = jnp.bitwise_xor(my, m)
            pl.semaphore_signal(barrier_sem, inc=1, device_id=(partner,),
                                device_id_type=pl.DeviceIdType.MESH)

        xb = x_ref[:, :].astype(jnp.bfloat16)
        scores = jnp.dot(xb, rw_ref[:, :].astype(jnp.bfloat16),
                         preferred_element_type=jnp.float32)
        s_max = jnp.max(scores, axis=-1, keepdims=True)
        p = jnp.exp(scores - s_max)
        probs = p / jnp.sum(p, axis=-1, keepdims=True)

        col = lax.broadcasted_iota(jnp.int32, (N_TOK, N_EXP), 1)
        routed = col == idx_ref[:, :]

        partial = jnp.zeros((N_TOK, H), jnp.float32)
        for k in range(E_PER):
            e = my * E_PER + k
            w = jnp.sum(jnp.where(routed & (col == e), probs, 0.0),
                        axis=1, keepdims=True)
            ye = jnp.dot(xb, ew_ref[k].astype(jnp.bfloat16),
                         preferred_element_type=jnp.float32)
            partial = partial + w * ye
        pb16 = partial.astype(jnp.bfloat16)
        acc_a[0] = pb16[:, :H_HALF]
        acc_b[0] = pb16[:, H_HALF:]

        pl.semaphore_wait(barrier_sem, N_STAGES)

        rdmas = []
        for s in range(N_STAGES):
            ra = pltpu.make_async_remote_copy(
                src_ref=acc_a.at[s],
                dst_ref=comm_a.at[s],
                send_sem=send_a.at[s],
                recv_sem=recv_a.at[s],
                device_id=(jnp.bitwise_xor(my, MASKS_A[s]),),
                device_id_type=pl.DeviceIdType.MESH,
            )
            rb = pltpu.make_async_remote_copy(
                src_ref=acc_b.at[s],
                dst_ref=comm_b.at[s],
                send_sem=send_b.at[s],
                recv_sem=recv_b.at[s],
                device_id=(jnp.bitwise_xor(my, MASKS_B[s]),),
                device_id_type=pl.DeviceIdType.MESH,
            )
            ra.start()
            rb.start()
            rdmas += [ra, rb]
            if s == 0:
                shared = jnp.dot(xb, sw_ref[:, :].astype(jnp.bfloat16),
                                 preferred_element_type=jnp.float32)
            ra.wait_recv()
            acc_a[s + 1] = acc_a[s] + comm_a[s]
            rb.wait_recv()
            acc_b[s + 1] = acc_b[s] + comm_b[s]

        out_ref[:, :H_HALF] = (acc_a[N_STAGES].astype(jnp.float32)
                               + shared[:, :H_HALF])
        out_ref[:, H_HALF:] = (acc_b[N_STAGES].astype(jnp.float32)
                               + shared[:, H_HALF:])
        for rdma in rdmas:
            rdma.wait_send()

    return pl.pallas_call(
        body,
        out_shape=jax.ShapeDtypeStruct((N_TOK, H), jnp.float32),
        in_specs=[pl.BlockSpec(memory_space=pltpu.VMEM)] * 5,
        out_specs=pl.BlockSpec(memory_space=pltpu.VMEM),
        scratch_shapes=[
            pltpu.VMEM((N_STAGES + 1, N_TOK, H_HALF), jnp.bfloat16),
            pltpu.VMEM((N_STAGES + 1, N_TOK, H_HALF), jnp.bfloat16),
            pltpu.VMEM((N_STAGES, N_TOK, H_HALF), jnp.bfloat16),
            pltpu.VMEM((N_STAGES, N_TOK, H_HALF), jnp.bfloat16),
            pltpu.SemaphoreType.DMA((N_STAGES,)),
            pltpu.SemaphoreType.DMA((N_STAGES,)),
            pltpu.SemaphoreType.DMA((N_STAGES,)),
            pltpu.SemaphoreType.DMA((N_STAGES,)),
        ],
        compiler_params=pltpu.CompilerParams(collective_id=0),
    )(x, router_W, route_idx, expert_W, shared_W)


# device time: 22342 ns/iter; 1.6665x vs baseline; 1.0644x over previous
import jax
import jax.numpy as jnp
from jax import lax
from jax.experimental import pallas as pl
from jax.experimental.pallas import tpu as pltpu

N_DEV = 32
N_STAGES = 5
MASKS_A = (1, 3, 8, 4, 16)
MASKS_B = (3, 8, 1, 16, 4)
E_PER = 2
N_TOK = 256
N_EXP = 64
H = 256
H_HALF = H // 2


def kernel(x, router_W, route_idx, expert_W, shared_W):
    def body(x_ref, rw_ref, idx_ref, ew_ref, sw_ref, out_ref,
             acc_a, acc_b, comm_a, comm_b,
             send_a, recv_a, send_b, recv_b):
        my = lax.axis_index("i")

        barrier_sem = pltpu.get_barrier_semaphore()
        for m in MASKS_A:
            partner = jnp.bitwise_xor(my, m)
            pl.semaphore_signal(barrier_sem, inc=1, device_id=(partner,),
                                device_id_type=pl.DeviceIdType.MESH)

        xb = x_ref[:, :].astype(jnp.bfloat16)
        scores = jnp.dot(xb, rw_ref[:, :].astype(jnp.bfloat16),
                         preferred_element_type=jnp.float32)
        s_max = jnp.max(scores, axis=-1, keepdims=True)
        p = jnp.exp(scores - s_max)
        probs = p / jnp.sum(p, axis=-1, keepdims=True)

        col = lax.broadcasted_iota(jnp.int32, (N_TOK, N_EXP), 1)
        routed = col == idx_ref[:, :]

        partial = jnp.zeros((N_TOK, H), jnp.float32)
        for k in range(E_PER):
            e = my * E_PER + k
            w = jnp.sum(jnp.where(routed & (col == e), probs, 0.0),
                        axis=1, keepdims=True)
            ye = jnp.dot(xb, ew_ref[k].astype(jnp.bfloat16),
                         preferred_element_type=jnp.float32)
            partial = partial + w * ye
        pb16 = partial.astype(jnp.bfloat16)
        acc_a[0] = pb16[:, :H_HALF]
        acc_b[0] = pb16[:, H_HALF:]

        pl.semaphore_wait(barrier_sem, N_STAGES)

        rdmas = []
        for s in range(N_STAGES):
            ra = pltpu.make_async_remote_copy(
                src_ref=acc_a.at[s],
                dst_ref=comm_a.at[s],
                send_sem=send_a.at[s],
                recv_sem=recv_a.at[s],
                device_id=(jnp.bitwise_xor(my, MASKS_A[s]),),
                device_id_type=pl.DeviceIdType.MESH,
            )
            rb = pltpu.make_async_remote_copy(
                src_ref=acc_b.at[s],
                dst_ref=comm_b.at[s],
                send_sem=send_b.at[s],
                recv_sem=recv_b.at[s],
                device_id=(jnp.bitwise_xor(my, MASKS_B[s]),),
                device_id_type=pl.DeviceIdType.MESH,
            )
            ra.start()
            rb.start()
            rdmas += [ra, rb]
            if s == 0:
                shared = jnp.dot(xb, sw_ref[:, :].astype(jnp.bfloat16),
                                 preferred_element_type=jnp.float32)
            ra.wait_recv()
            if s < N_STAGES - 1:
                acc_a[s + 1] = acc_a[s] + comm_a[s]
            else:
                out_ref[:, :H_HALF] = (
                    acc_a[s].astype(jnp.float32)
                    + comm_a[s].astype(jnp.float32)
                    + shared[:, :H_HALF])
            rb.wait_recv()
            if s < N_STAGES - 1:
                acc_b[s + 1] = acc_b[s] + comm_b[s]
            else:
                out_ref[:, H_HALF:] = (
                    acc_b[s].astype(jnp.float32)
                    + comm_b[s].astype(jnp.float32)
                    + shared[:, H_HALF:])
        for rdma in rdmas:
            rdma.wait_send()

    return pl.pallas_call(
        body,
        out_shape=jax.ShapeDtypeStruct((N_TOK, H), jnp.float32),
        in_specs=[pl.BlockSpec(memory_space=pltpu.VMEM)] * 5,
        out_specs=pl.BlockSpec(memory_space=pltpu.VMEM),
        scratch_shapes=[
            pltpu.VMEM((N_STAGES + 1, N_TOK, H_HALF), jnp.bfloat16),
            pltpu.VMEM((N_STAGES + 1, N_TOK, H_HALF), jnp.bfloat16),
            pltpu.VMEM((N_STAGES, N_TOK, H_HALF), jnp.bfloat16),
            pltpu.VMEM((N_STAGES, N_TOK, H_HALF), jnp.bfloat16),
            pltpu.SemaphoreType.DMA((N_STAGES,)),
            pltpu.SemaphoreType.DMA((N_STAGES,)),
            pltpu.SemaphoreType.DMA((N_STAGES,)),
            pltpu.SemaphoreType.DMA((N_STAGES,)),
        ],
        compiler_params=pltpu.CompilerParams(collective_id=0),
    )(x, router_W, route_idx, expert_W, shared_W)
